# Initial kernel scaffold; baseline (speedup 1.0000x reference)
import jax
import jax.numpy as jnp
from jax.experimental import pallas as pl


def kernel(x, edge_index, W1, b1, W2, b2, prelu_a, lin1_W, lin1_b, bn_g, bn_b, prelu2_a, out_W, out_b):
    raise NotImplementedError("write your pallas kernel here")


# speedup vs baseline: 3.6641x; 3.6641x over previous
import functools

import jax
import jax.numpy as jnp
from jax import lax
from jax.experimental import pallas as pl
from jax.experimental.pallas import tpu as pltpu
from jax.experimental.pallas import tpu_sc as plsc

F32 = jnp.float32
I32 = jnp.int32

N = 10000
E = 320000
D_IN = 128
D_H = 256

NPAD = 10240
NT = 32
ROWS_PT = NPAD // NT
DUMMY = ROWS_PT
ACC_R = ROWS_PT + 4

SCAN_C = 512
FLUSH = 1024
LCAP = FLUSH + 144
CAP = (E // FLUSH + 1) * FLUSH + FLUSH
G = 64

GB = NPAD // 8
GRID = 8

HIST = 336


def _wid():
    return lax.axis_index("s") * 2 + lax.axis_index("c")


@functools.lru_cache(maxsize=None)
def _sc_route_kernel():
    mesh = plsc.VectorSubcoreMesh(core_axis_name="c", subcore_axis_name="s")

    @functools.partial(
        pl.kernel,
        compiler_params=pltpu.CompilerParams(needs_layout_passes=False),
        out_type=[
            jax.ShapeDtypeStruct((NT * CAP,), I32),
            jax.ShapeDtypeStruct((NT * CAP,), I32),
            jax.ShapeDtypeStruct((NT, 16), I32),
            jax.ShapeDtypeStruct((NPAD,), I32),
        ],
        mesh=mesh,
        scratch_types=[
            pltpu.VMEM((SCAN_C,), I32),
            pltpu.VMEM((SCAN_C,), I32),
            pltpu.VMEM((LCAP,), I32),
            pltpu.VMEM((LCAP,), I32),
            pltpu.VMEM((HIST,), I32),
            pltpu.VMEM((16,), I32),
        ],
    )
    def route(src_hbm, dst_hbm, ldst_out, src_out, cnt_out, deg_out,
              dstv, srcv, lbuf, sbuf, degl, cntv):
        w = _wid()
        lo = w * ROWS_PT
        wbase = w * CAP
        lanes = lax.iota(I32, 16)
        dummy_l = jnp.full((16,), DUMMY, I32)
        dummy_s = lo + lanes
        zero16 = jnp.zeros((16,), I32)
        for k in range(LCAP // 16):
            lbuf[pl.ds(k * 16, 16)] = dummy_l
            sbuf[pl.ds(k * 16, 16)] = dummy_s
        for k in range(HIST // 16):
            degl[pl.ds(k * 16, 16)] = zero16

        def chunk(i, carry):
            cnt, written = carry
            off = pl.multiple_of(i * SCAN_C, SCAN_C)
            pltpu.sync_copy(dst_hbm.at[pl.ds(off, SCAN_C)], dstv)
            pltpu.sync_copy(src_hbm.at[pl.ds(off, SCAN_C)], srcv)
            for j in range(SCAN_C // 16):
                d16 = dstv[pl.ds(j * 16, 16)]
                s16 = srcv[pl.ds(j * 16, 16)]
                l16 = d16 - lo
                m = jnp.logical_and(l16 >= 0, l16 < ROWS_PT)
                mi = m.astype(I32)
                pos = cnt + plsc.cumsum(mi) - 1
                plsc.store_scatter(lbuf, [pos], l16, mask=m)
                plsc.store_scatter(sbuf, [pos], s16, mask=m)
                c16, last = plsc.scan_count(l16, m)
                plsc.addupdate_scatter(degl, [l16], c16, mask=last)
                cnt = cnt + jnp.sum(mi)

            do_flush = cnt >= FLUSH

            @pl.when(do_flush)
            def _():
                fo = pl.multiple_of(wbase + written, FLUSH)
                pltpu.sync_copy(lbuf.at[pl.ds(0, FLUSH)],
                                ldst_out.at[pl.ds(fo, FLUSH)])
                pltpu.sync_copy(sbuf.at[pl.ds(0, FLUSH)],
                                src_out.at[pl.ds(fo, FLUSH)])
                for k in range(9):
                    lbuf[pl.ds(k * 16, 16)] = lbuf[pl.ds(FLUSH + k * 16, 16)]
                    sbuf[pl.ds(k * 16, 16)] = sbuf[pl.ds(FLUSH + k * 16, 16)]
                for k in range(9, LCAP // 16):
                    lbuf[pl.ds(k * 16, 16)] = dummy_l
                    sbuf[pl.ds(k * 16, 16)] = dummy_s

            cnt = jnp.where(do_flush, cnt - FLUSH, cnt)
            written = jnp.where(do_flush, written + FLUSH, written)
            return cnt, written

        cnt, written = lax.fori_loop(0, E // SCAN_C, chunk, (0, 0))
        fo = pl.multiple_of(wbase + written, FLUSH)
        pltpu.sync_copy(lbuf.at[pl.ds(0, FLUSH)],
                        ldst_out.at[pl.ds(fo, FLUSH)])
        pltpu.sync_copy(sbuf.at[pl.ds(0, FLUSH)],
                        src_out.at[pl.ds(fo, FLUSH)])
        cntv[...] = jnp.full((16,), written + FLUSH, I32)
        pltpu.sync_copy(cntv, cnt_out.at[w])
        pltpu.sync_copy(degl.at[pl.ds(0, ROWS_PT)],
                        deg_out.at[pl.ds(pl.multiple_of(lo, ROWS_PT),
                                         ROWS_PT)])

    return route


@functools.lru_cache(maxsize=None)
def _sc_aggregate_kernel():
    mesh = plsc.VectorSubcoreMesh(core_axis_name="c", subcore_axis_name="s")

    @functools.partial(
        pl.kernel,
        compiler_params=pltpu.CompilerParams(needs_layout_passes=False),
        out_type=jax.ShapeDtypeStruct((NPAD, D_H), F32),
        mesh=mesh,
        scratch_types=[
            pltpu.VMEM((G + 16,), I32),
            pltpu.VMEM((G,), I32),
            pltpu.VMEM((G, D_H), F32),
            pltpu.VMEM((ACC_R, D_H), F32),
            pltpu.VMEM((16,), I32),
            pltpu.SemaphoreType.DMA,
        ],
    )
    def agg(ldst_hbm, src_hbm, cnt_hbm, t_hbm, out_hbm,
            ldstv, srcv, rows, acc, cntv, sem):
        w = _wid()
        lo = w * ROWS_PT
        wbase = w * CAP
        pltpu.sync_copy(t_hbm.at[pl.ds(lo, ROWS_PT)],
                        acc.at[pl.ds(0, ROWS_PT)])
        pltpu.sync_copy(cnt_hbm.at[w], cntv)
        m = cntv[...][0]

        def chunk(c, carry):
            off = pl.multiple_of(wbase + c * G, G)
            pltpu.sync_copy(ldst_hbm.at[pl.ds(off, G)],
                            ldstv.at[pl.ds(0, G)])
            pltpu.sync_copy(src_hbm.at[pl.ds(off, G)], srcv)
            pltpu.async_copy(t_hbm.at[plsc.Indices(srcv)], rows, sem).wait()

            def row_add(g, carry2):
                d = ldstv[pl.ds(g, 16)][0]
                for j in range(D_H // 16):
                    sl = pl.ds(j * 16, 16)
                    plsc.addupdate(acc.at[d, sl], rows[g, sl])
                return carry2

            lax.fori_loop(0, G, row_add, 0)
            return carry

        lax.fori_loop(0, m // G, chunk, 0)
        pltpu.sync_copy(acc.at[pl.ds(0, ROWS_PT)],
                        out_hbm.at[pl.ds(lo, ROWS_PT)])

    return agg


def _prelu(v, a):
    return jnp.where(v >= 0, v, a * v)


def _dinv(deg_ref):
    return lax.rsqrt(deg_ref[...].astype(F32) + 1.0)


def _m1_body(x_ref, w_ref, deg_ref, o_ref):
    h = jnp.dot(x_ref[...], w_ref[...], preferred_element_type=F32)
    o_ref[...] = h * _dinv(deg_ref)


def _tc_scale_mm(xp, w1, deg):
    return pl.pallas_call(
        _m1_body,
        grid=(GRID,),
        in_specs=[
            pl.BlockSpec((GB, D_IN), lambda i: (i, 0)),
            pl.BlockSpec((D_IN, D_H), lambda i: (0, 0)),
            pl.BlockSpec((GB, 1), lambda i: (i, 0)),
        ],
        out_specs=pl.BlockSpec((GB, D_H), lambda i: (i, 0)),
        out_shape=jax.ShapeDtypeStruct((NPAD, D_H), F32),
    )(xp, w1, deg)


def _m2_body(raw_ref, deg_ref, xp_ref, w2_ref, b1_ref, a_ref, o_ref):
    dinv = _dinv(deg_ref)
    x1 = _prelu(raw_ref[...] * dinv + b1_ref[...], a_ref[0, 0])
    h2 = jnp.dot(x1, w2_ref[0:D_H, :], preferred_element_type=F32)
    h2 = h2 + jnp.dot(xp_ref[...], w2_ref[D_H:D_H + D_IN, :],
                      preferred_element_type=F32)
    o_ref[...] = h2 * dinv


def _tc_layer2(raw1, deg, xp, w2, b1, a):
    return pl.pallas_call(
        _m2_body,
        grid=(GRID,),
        in_specs=[
            pl.BlockSpec((GB, D_H), lambda i: (i, 0)),
            pl.BlockSpec((GB, 1), lambda i: (i, 0)),
            pl.BlockSpec((GB, D_IN), lambda i: (i, 0)),
            pl.BlockSpec((D_H + D_IN, D_H), lambda i: (0, 0)),
            pl.BlockSpec((1, D_H), lambda i: (0, 0)),
            pl.BlockSpec((1, 1), lambda i: (0, 0)),
        ],
        out_specs=pl.BlockSpec((GB, D_H), lambda i: (i, 0)),
        out_shape=jax.ShapeDtypeStruct((NPAD, D_H), F32),
    )(raw1, deg, xp, w2, b1, a)


def _m3_body(raw_ref, deg_ref, b2_ref, a_ref, w_ref, lb_ref, h_ref, st_ref):
    i = pl.program_id(0)
    x2 = _prelu(raw_ref[...] * _dinv(deg_ref) + b2_ref[...], a_ref[0, 0])
    h = jnp.dot(x2, w_ref[...], preferred_element_type=F32) + lb_ref[...]
    h_ref[...] = h
    rows = i * GB + lax.broadcasted_iota(I32, (GB, 1), 0)
    hm = h * (rows < N).astype(F32)
    st = jnp.concatenate(
        [jnp.sum(hm, axis=0, keepdims=True),
         jnp.sum(hm * hm, axis=0, keepdims=True),
         jnp.zeros((6, D_H), F32)], axis=0)

    @pl.when(i == 0)
    def _():
        st_ref[...] = jnp.zeros_like(st_ref)

    st_ref[...] += st


def _tc_lin1(raw2, deg, b2, a, lin1_w, lin1_b):
    return pl.pallas_call(
        _m3_body,
        grid=(GRID,),
        in_specs=[
            pl.BlockSpec((GB, D_H), lambda i: (i, 0)),
            pl.BlockSpec((GB, 1), lambda i: (i, 0)),
            pl.BlockSpec((1, D_H), lambda i: (0, 0)),
            pl.BlockSpec((1, 1), lambda i: (0, 0)),
            pl.BlockSpec((D_H, D_H), lambda i: (0, 0)),
            pl.BlockSpec((1, D_H), lambda i: (0, 0)),
        ],
        out_specs=[
            pl.BlockSpec((GB, D_H), lambda i: (i, 0)),
            pl.BlockSpec((8, D_H), lambda i: (0, 0)),
        ],
        out_shape=[
            jax.ShapeDtypeStruct((NPAD, D_H), F32),
            jax.ShapeDtypeStruct((8, D_H), F32),
        ],
    )(raw2, deg, b2, a, lin1_w, lin1_b)


def _m4_body(h_ref, st_ref, g_ref, b_ref, a_ref, ow_ref, ob_ref, o_ref):
    inv_n = 1.0 / N
    m = st_ref[0:1, :] * inv_n
    v = st_ref[1:2, :] * inv_n - m * m
    hn = (h_ref[...] - m) * lax.rsqrt(v + 1e-5) * g_ref[...] + b_ref[...]
    hp = _prelu(hn, a_ref[0, 0])
    o_ref[...] = jnp.dot(hp, ow_ref[...], preferred_element_type=F32) \
        + ob_ref[0, 0]


def _tc_head(h, st, bn_g, bn_b, a2, out_w, out_b):
    return pl.pallas_call(
        _m4_body,
        grid=(GRID,),
        in_specs=[
            pl.BlockSpec((GB, D_H), lambda i: (i, 0)),
            pl.BlockSpec((8, D_H), lambda i: (0, 0)),
            pl.BlockSpec((1, D_H), lambda i: (0, 0)),
            pl.BlockSpec((1, D_H), lambda i: (0, 0)),
            pl.BlockSpec((1, 1), lambda i: (0, 0)),
            pl.BlockSpec((D_H, 1), lambda i: (0, 0)),
            pl.BlockSpec((1, 1), lambda i: (0, 0)),
        ],
        out_specs=pl.BlockSpec((GB, 1), lambda i: (i, 0)),
        out_shape=jax.ShapeDtypeStruct((NPAD, 1), F32),
    )(h, st, bn_g, bn_b, a2, out_w, out_b)


def kernel(x, edge_index, W1, b1, W2, b2, prelu_a, lin1_W, lin1_b, bn_g,
           bn_b, prelu2_a, out_W, out_b):
    src = edge_index[0]
    dst = edge_index[1]
    xp = jnp.pad(x, ((0, NPAD - N), (0, 0)))
    b1r = jnp.reshape(b1, (1, D_H))
    b2r = jnp.reshape(b2, (1, D_H))
    ar = jnp.reshape(prelu_a, (1, 1))
    a2r = jnp.reshape(prelu2_a, (1, 1))

    ldst_list, src_list, cnts, deg = _sc_route_kernel()(src, dst)
    deg2 = jnp.reshape(deg, (NPAD, 1))
    t1 = _tc_scale_mm(xp, W1, deg2)
    agg = _sc_aggregate_kernel()
    r1 = agg(ldst_list, src_list, cnts, t1)
    t2 = _tc_layer2(r1, deg2, xp, W2, b1r, ar)
    r2 = agg(ldst_list, src_list, cnts, t2)
    h, st = _tc_lin1(r2, deg2, b2r, ar, lin1_W, jnp.reshape(lin1_b, (1, D_H)))
    out = _tc_head(h, st, jnp.reshape(bn_g, (1, D_H)),
                   jnp.reshape(bn_b, (1, D_H)), a2r, out_W,
                   jnp.reshape(out_b, (1, 1)))
    return out[:N]


# speedup vs baseline: 4.8656x; 1.3279x over previous
import functools

import jax
import jax.numpy as jnp
from jax import lax
from jax.experimental import pallas as pl
from jax.experimental.pallas import tpu as pltpu
from jax.experimental.pallas import tpu_sc as plsc

F32 = jnp.float32
I32 = jnp.int32

N = 10000
E = 320000
D_IN = 128
D_H = 256

NPAD = 10240
NT = 32
ROWS_PT = NPAD // NT
DUMMY = ROWS_PT
ACC_R = ROWS_PT + 4

SCAN_C = 1280
FLUSH = 1024
LCAP = FLUSH + 144
CAP = (E // FLUSH + 1) * FLUSH + FLUSH
G = 64

GB = NPAD // 8
GRID = 8

HIST = 336


def _wid():
    return lax.axis_index("s") * 2 + lax.axis_index("c")


@functools.lru_cache(maxsize=None)
def _sc_route_kernel():
    mesh = plsc.VectorSubcoreMesh(core_axis_name="c", subcore_axis_name="s")

    @functools.partial(
        pl.kernel,
        compiler_params=pltpu.CompilerParams(needs_layout_passes=False),
        out_type=[
            jax.ShapeDtypeStruct((NT * CAP,), I32),
            jax.ShapeDtypeStruct((NT * CAP,), I32),
            jax.ShapeDtypeStruct((NT, 16), I32),
            jax.ShapeDtypeStruct((NPAD,), I32),
        ],
        mesh=mesh,
        scratch_types=[
            pltpu.VMEM((SCAN_C,), I32),
            pltpu.VMEM((SCAN_C,), I32),
            pltpu.VMEM((SCAN_C,), I32),
            pltpu.VMEM((SCAN_C,), I32),
            pltpu.VMEM((LCAP,), I32),
            pltpu.VMEM((LCAP,), I32),
            pltpu.VMEM((HIST,), I32),
            pltpu.VMEM((16,), I32),
            pltpu.SemaphoreType.DMA,
            pltpu.SemaphoreType.DMA,
            pltpu.SemaphoreType.DMA,
            pltpu.SemaphoreType.DMA,
        ],
    )
    def route(src_hbm, dst_hbm, ldst_out, src_out, cnt_out, deg_out,
              dstva, srcva, dstvb, srcvb, lbuf, sbuf, degl, cntv,
              sda, ssa, sdb, ssb):
        w = _wid()
        lo = w * ROWS_PT
        wbase = w * CAP
        lanes = lax.iota(I32, 16)
        dummy_l = jnp.full((16,), DUMMY, I32)
        dummy_s = lo + lanes
        zero16 = jnp.zeros((16,), I32)
        for k in range(LCAP // 16):
            lbuf[pl.ds(k * 16, 16)] = dummy_l
            sbuf[pl.ds(k * 16, 16)] = dummy_s
        for k in range(HIST // 16):
            degl[pl.ds(k * 16, 16)] = zero16

        nchunks = E // SCAN_C

        def load(i, dv, sv, sd, ss):
            off = pl.multiple_of(i * SCAN_C, SCAN_C)
            pltpu.async_copy(dst_hbm.at[pl.ds(off, SCAN_C)], dv, sd)
            pltpu.async_copy(src_hbm.at[pl.ds(off, SCAN_C)], sv, ss)

        def wait(dv, sv, sd, ss):
            pltpu.make_async_copy(dst_hbm.at[pl.ds(0, SCAN_C)], dv, sd).wait()
            pltpu.make_async_copy(src_hbm.at[pl.ds(0, SCAN_C)], sv, ss).wait()

        def process(dv, sv, cnt, written):
            for j in range(SCAN_C // 16):
                d16 = dv[pl.ds(j * 16, 16)]
                s16 = sv[pl.ds(j * 16, 16)]
                l16 = d16 - lo
                m = jnp.logical_and(l16 >= 0, l16 < ROWS_PT)
                mi = m.astype(I32)
                pos = cnt + plsc.cumsum(mi) - 1
                plsc.store_scatter(lbuf, [pos], l16, mask=m)
                plsc.store_scatter(sbuf, [pos], s16, mask=m)
                c16, last = plsc.scan_count(l16, m)
                plsc.addupdate_scatter(degl, [l16], c16, mask=last)
                cnt = cnt + jnp.sum(mi)

            do_flush = cnt >= FLUSH

            @pl.when(do_flush)
            def _():
                fo = pl.multiple_of(wbase + written, FLUSH)
                pltpu.sync_copy(lbuf.at[pl.ds(0, FLUSH)],
                                ldst_out.at[pl.ds(fo, FLUSH)])
                pltpu.sync_copy(sbuf.at[pl.ds(0, FLUSH)],
                                src_out.at[pl.ds(fo, FLUSH)])
                for k in range(9):
                    lbuf[pl.ds(k * 16, 16)] = lbuf[pl.ds(FLUSH + k * 16, 16)]
                    sbuf[pl.ds(k * 16, 16)] = sbuf[pl.ds(FLUSH + k * 16, 16)]
                for k in range(9, LCAP // 16):
                    lbuf[pl.ds(k * 16, 16)] = dummy_l
                    sbuf[pl.ds(k * 16, 16)] = dummy_s

            cnt = jnp.where(do_flush, cnt - FLUSH, cnt)
            written = jnp.where(do_flush, written + FLUSH, written)
            return cnt, written

        load(0, dstva, srcva, sda, ssa)

        def chunk_pair(p, carry):
            cnt, written = carry
            i = p * 2
            load(i + 1, dstvb, srcvb, sdb, ssb)
            wait(dstva, srcva, sda, ssa)
            cnt, written = process(dstva, srcva, cnt, written)

            @pl.when(i + 2 < nchunks)
            def _():
                load(i + 2, dstva, srcva, sda, ssa)

            wait(dstvb, srcvb, sdb, ssb)
            cnt, written = process(dstvb, srcvb, cnt, written)
            return cnt, written

        cnt, written = lax.fori_loop(0, nchunks // 2, chunk_pair, (0, 0))
        fo = pl.multiple_of(wbase + written, FLUSH)
        pltpu.sync_copy(lbuf.at[pl.ds(0, FLUSH)],
                        ldst_out.at[pl.ds(fo, FLUSH)])
        pltpu.sync_copy(sbuf.at[pl.ds(0, FLUSH)],
                        src_out.at[pl.ds(fo, FLUSH)])
        cntv[...] = jnp.full((16,), written + FLUSH, I32)
        pltpu.sync_copy(cntv, cnt_out.at[w])
        pltpu.sync_copy(degl.at[pl.ds(0, ROWS_PT)],
                        deg_out.at[pl.ds(pl.multiple_of(lo, ROWS_PT),
                                         ROWS_PT)])

    return route


@functools.lru_cache(maxsize=None)
def _sc_aggregate_kernel():
    mesh = plsc.VectorSubcoreMesh(core_axis_name="c", subcore_axis_name="s")

    @functools.partial(
        pl.kernel,
        compiler_params=pltpu.CompilerParams(needs_layout_passes=False),
        out_type=jax.ShapeDtypeStruct((NPAD, D_H), F32),
        mesh=mesh,
        scratch_types=[
            pltpu.VMEM((G + 16,), I32),
            pltpu.VMEM((G,), I32),
            pltpu.VMEM((G, D_H), F32),
            pltpu.VMEM((G + 16,), I32),
            pltpu.VMEM((G,), I32),
            pltpu.VMEM((G, D_H), F32),
            pltpu.VMEM((ACC_R, D_H), F32),
            pltpu.VMEM((16,), I32),
            pltpu.SemaphoreType.DMA,
            pltpu.SemaphoreType.DMA,
        ],
    )
    def agg(ldst_hbm, src_hbm, cnt_hbm, t_hbm, out_hbm,
            ldstva, srcva, rowsa, ldstvb, srcvb, rowsb, acc, cntv,
            sema, semb):
        w = _wid()
        lo = w * ROWS_PT
        wbase = w * CAP
        pltpu.sync_copy(t_hbm.at[pl.ds(lo, ROWS_PT)],
                        acc.at[pl.ds(0, ROWS_PT)])
        pltpu.sync_copy(cnt_hbm.at[w], cntv)
        m = cntv[...][0]
        npairs = m // (2 * G)

        def start(c, ldstv, srcv, rows, sem):
            off = pl.multiple_of(wbase + c * G, G)
            pltpu.sync_copy(ldst_hbm.at[pl.ds(off, G)],
                            ldstv.at[pl.ds(0, G)])
            pltpu.sync_copy(src_hbm.at[pl.ds(off, G)], srcv)
            pltpu.async_copy(t_hbm.at[plsc.Indices(srcv)], rows, sem)

        def accumulate(ldstv, srcv, rows, sem):
            pltpu.make_async_copy(t_hbm.at[plsc.Indices(srcv)], rows,
                                  sem).wait()

            def row_add(g, carry2):
                d = ldstv[pl.ds(g, 16)][0]
                for j in range(D_H // 16):
                    sl = pl.ds(j * 16, 16)
                    plsc.addupdate(acc.at[d, sl], rows[g, sl])
                return carry2

            lax.fori_loop(0, G, row_add, 0)

        @pl.when(npairs > 0)
        def _():
            start(0, ldstva, srcva, rowsa, sema)

        def chunk_pair(p, carry):
            c = p * 2
            start(c + 1, ldstvb, srcvb, rowsb, semb)
            accumulate(ldstva, srcva, rowsa, sema)

            @pl.when(c + 2 < npairs * 2)
            def _():
                start(c + 2, ldstva, srcva, rowsa, sema)

            accumulate(ldstvb, srcvb, rowsb, semb)
            return carry

        lax.fori_loop(0, npairs, chunk_pair, 0)
        pltpu.sync_copy(acc.at[pl.ds(0, ROWS_PT)],
                        out_hbm.at[pl.ds(lo, ROWS_PT)])

    return agg


def _prelu(v, a):
    return jnp.where(v >= 0, v, a * v)


def _dinv(deg_ref):
    return lax.rsqrt(deg_ref[...].astype(F32) + 1.0)


def _m1_body(x_ref, w_ref, deg_ref, o_ref):
    h = jnp.dot(x_ref[...], w_ref[...], preferred_element_type=F32)
    o_ref[...] = h * _dinv(deg_ref)


def _tc_scale_mm(xp, w1, deg):
    return pl.pallas_call(
        _m1_body,
        grid=(GRID,),
        in_specs=[
            pl.BlockSpec((GB, D_IN), lambda i: (i, 0)),
            pl.BlockSpec((D_IN, D_H), lambda i: (0, 0)),
            pl.BlockSpec((GB, 1), lambda i: (i, 0)),
        ],
        out_specs=pl.BlockSpec((GB, D_H), lambda i: (i, 0)),
        out_shape=jax.ShapeDtypeStruct((NPAD, D_H), F32),
    )(xp, w1, deg)


def _m2_body(raw_ref, deg_ref, xp_ref, w2_ref, b1_ref, a_ref, o_ref):
    dinv = _dinv(deg_ref)
    x1 = _prelu(raw_ref[...] * dinv + b1_ref[...], a_ref[0, 0])
    h2 = jnp.dot(x1, w2_ref[0:D_H, :], preferred_element_type=F32)
    h2 = h2 + jnp.dot(xp_ref[...], w2_ref[D_H:D_H + D_IN, :],
                      preferred_element_type=F32)
    o_ref[...] = h2 * dinv


def _tc_layer2(raw1, deg, xp, w2, b1, a):
    return pl.pallas_call(
        _m2_body,
        grid=(GRID,),
        in_specs=[
            pl.BlockSpec((GB, D_H), lambda i: (i, 0)),
            pl.BlockSpec((GB, 1), lambda i: (i, 0)),
            pl.BlockSpec((GB, D_IN), lambda i: (i, 0)),
            pl.BlockSpec((D_H + D_IN, D_H), lambda i: (0, 0)),
            pl.BlockSpec((1, D_H), lambda i: (0, 0)),
            pl.BlockSpec((1, 1), lambda i: (0, 0)),
        ],
        out_specs=pl.BlockSpec((GB, D_H), lambda i: (i, 0)),
        out_shape=jax.ShapeDtypeStruct((NPAD, D_H), F32),
    )(raw1, deg, xp, w2, b1, a)


def _m3_body(raw_ref, deg_ref, b2_ref, a_ref, w_ref, lb_ref, h_ref, st_ref):
    i = pl.program_id(0)
    x2 = _prelu(raw_ref[...] * _dinv(deg_ref) + b2_ref[...], a_ref[0, 0])
    h = jnp.dot(x2, w_ref[...], preferred_element_type=F32) + lb_ref[...]
    h_ref[...] = h
    rows = i * GB + lax.broadcasted_iota(I32, (GB, 1), 0)
    hm = h * (rows < N).astype(F32)
    st = jnp.concatenate(
        [jnp.sum(hm, axis=0, keepdims=True),
         jnp.sum(hm * hm, axis=0, keepdims=True),
         jnp.zeros((6, D_H), F32)], axis=0)

    @pl.when(i == 0)
    def _():
        st_ref[...] = jnp.zeros_like(st_ref)

    st_ref[...] += st


def _tc_lin1(raw2, deg, b2, a, lin1_w, lin1_b):
    return pl.pallas_call(
        _m3_body,
        grid=(GRID,),
        in_specs=[
            pl.BlockSpec((GB, D_H), lambda i: (i, 0)),
            pl.BlockSpec((GB, 1), lambda i: (i, 0)),
            pl.BlockSpec((1, D_H), lambda i: (0, 0)),
            pl.BlockSpec((1, 1), lambda i: (0, 0)),
            pl.BlockSpec((D_H, D_H), lambda i: (0, 0)),
            pl.BlockSpec((1, D_H), lambda i: (0, 0)),
        ],
        out_specs=[
            pl.BlockSpec((GB, D_H), lambda i: (i, 0)),
            pl.BlockSpec((8, D_H), lambda i: (0, 0)),
        ],
        out_shape=[
            jax.ShapeDtypeStruct((NPAD, D_H), F32),
            jax.ShapeDtypeStruct((8, D_H), F32),
        ],
    )(raw2, deg, b2, a, lin1_w, lin1_b)


def _m4_body(h_ref, st_ref, g_ref, b_ref, a_ref, ow_ref, ob_ref, o_ref):
    inv_n = 1.0 / N
    m = st_ref[0:1, :] * inv_n
    v = st_ref[1:2, :] * inv_n - m * m
    hn = (h_ref[...] - m) * lax.rsqrt(v + 1e-5) * g_ref[...] + b_ref[...]
    hp = _prelu(hn, a_ref[0, 0])
    o_ref[...] = jnp.dot(hp, ow_ref[...], preferred_element_type=F32) \
        + ob_ref[0, 0]


def _tc_head(h, st, bn_g, bn_b, a2, out_w, out_b):
    return pl.pallas_call(
        _m4_body,
        grid=(GRID,),
        in_specs=[
            pl.BlockSpec((GB, D_H), lambda i: (i, 0)),
            pl.BlockSpec((8, D_H), lambda i: (0, 0)),
            pl.BlockSpec((1, D_H), lambda i: (0, 0)),
            pl.BlockSpec((1, D_H), lambda i: (0, 0)),
            pl.BlockSpec((1, 1), lambda i: (0, 0)),
            pl.BlockSpec((D_H, 1), lambda i: (0, 0)),
            pl.BlockSpec((1, 1), lambda i: (0, 0)),
        ],
        out_specs=pl.BlockSpec((GB, 1), lambda i: (i, 0)),
        out_shape=jax.ShapeDtypeStruct((NPAD, 1), F32),
    )(h, st, bn_g, bn_b, a2, out_w, out_b)


def kernel(x, edge_index, W1, b1, W2, b2, prelu_a, lin1_W, lin1_b, bn_g,
           bn_b, prelu2_a, out_W, out_b):
    src = edge_index[0]
    dst = edge_index[1]
    xp = jnp.pad(x, ((0, NPAD - N), (0, 0)))
    b1r = jnp.reshape(b1, (1, D_H))
    b2r = jnp.reshape(b2, (1, D_H))
    ar = jnp.reshape(prelu_a, (1, 1))
    a2r = jnp.reshape(prelu2_a, (1, 1))

    ldst_list, src_list, cnts, deg = _sc_route_kernel()(src, dst)
    deg2 = jnp.reshape(deg, (NPAD, 1))
    t1 = _tc_scale_mm(xp, W1, deg2)
    agg = _sc_aggregate_kernel()
    r1 = agg(ldst_list, src_list, cnts, t1)
    t2 = _tc_layer2(r1, deg2, xp, W2, b1r, ar)
    r2 = agg(ldst_list, src_list, cnts, t2)
    h, st = _tc_lin1(r2, deg2, b2r, ar, lin1_W, jnp.reshape(lin1_b, (1, D_H)))
    out = _tc_head(h, st, jnp.reshape(bn_g, (1, D_H)),
                   jnp.reshape(bn_b, (1, D_H)), a2r, out_W,
                   jnp.reshape(out_b, (1, 1)))
    return out[:N]
